# baseline (speedup 1.0000x reference)
import functools

import jax
import jax.numpy as jnp
from jax import lax
from jax.experimental import pallas as pl
from jax.experimental.pallas import tpu as pltpu
from jax.experimental.pallas import tpu_sc as plsc

N_NODES = 10000
D_FEAT = 128
N_EDGES = 320000
DEPTH = 3

NC = 2
NS = 16
DH = D_FEAT // NC
CHUNK = 96
NCH = 212
EPT = CHUNK * NCH
E_PAD = EPT * NS
BLOCKS = NCH // 4
RPT = 632
RPT_LAST = N_NODES - (NS - 1) * RPT

_mesh = plsc.VectorSubcoreMesh(
    core_axis_name="c", subcore_axis_name="s", num_cores=NC, num_subcores=NS
)


@functools.partial(
    pl.kernel,
    out_type=jax.ShapeDtypeStruct((DEPTH, NC, N_NODES, DH), jnp.float32),
    mesh=_mesh,
    scratch_types=[
        pltpu.VMEM_SHARED((N_NODES, DH), jnp.float32),
        pltpu.VMEM_SHARED((N_NODES, DH), jnp.float32),
        pltpu.VMEM((2, CHUNK), jnp.int32),
        pltpu.VMEM((2, CHUNK), jnp.int32),
        pltpu.VMEM((2, CHUNK), jnp.int32),
        pltpu.VMEM((2, CHUNK), jnp.int32),
        pltpu.VMEM((CHUNK,), jnp.float32),
        pltpu.VMEM((CHUNK,), jnp.float32),
        pltpu.VMEM((CHUNK,), jnp.float32),
        pltpu.VMEM((CHUNK,), jnp.float32),
        pltpu.VMEM((CHUNK, DH), jnp.float32),
        pltpu.VMEM((CHUNK, DH), jnp.float32),
        pltpu.VMEM((CHUNK, DH), jnp.float32),
        pltpu.VMEM((CHUNK, DH), jnp.float32),
        pltpu.VMEM((DEPTH, 16), jnp.float32),
        pltpu.SemaphoreType.DMA,
        pltpu.SemaphoreType.DMA,
        pltpu.SemaphoreType.DMA,
        pltpu.SemaphoreType.DMA,
        pltpu.SemaphoreType.DMA,
        pltpu.SemaphoreType.DMA,
        pltpu.SemaphoreType.DMA,
        pltpu.SemaphoreType.DMA,
        pltpu.SemaphoreType.DMA,
        pltpu.SemaphoreType.DMA,
        pltpu.SemaphoreType.DMA,
        pltpu.SemaphoreType.DMA,
    ],
    compiler_params=pltpu.CompilerParams(use_tc_tiling_on_sc=False),
)
def _poly_conv(xp_hbm, edges_hbm, w_hbm, alphas_hbm, zeros_hbm, y_hbm,
               bufa, bufb, eb0, eb1, eb2, eb3, wb0, wb1, wb2, wb3,
               rb0, rb1, rb2, rb3, alphas_v,
               se0, se1, se2, se3, sg0, sg1, sg2, sg3, ss0, ss1, ss2, ss3):
    c = lax.axis_index("c")
    s = lax.axis_index("s")
    rbase = pl.multiple_of(s * RPT, 8)

    ebufs = (eb0, eb1, eb2, eb3)
    wbufs = (wb0, wb1, wb2, wb3)
    rbufs = (rb0, rb1, rb2, rb3)
    se = (se0, se1, se2, se3)
    sg = (sg0, sg1, sg2, sg3)
    ss = (ss0, ss1, ss2, ss3)

    def each_slice(fn):
        @pl.when(s < NS - 1)
        def _main():
            fn(rbase, RPT)

        @pl.when(s == NS - 1)
        def _last():
            fn(rbase, RPT_LAST)

    each_slice(lambda b, n: pltpu.sync_copy(xp_hbm.at[c, pl.ds(b, n)],
                                            bufa.at[pl.ds(b, n)]))
    each_slice(lambda b, n: pltpu.sync_copy(zeros_hbm.at[pl.ds(b, n)],
                                            bufb.at[pl.ds(b, n)]))
    pltpu.sync_copy(alphas_hbm, alphas_v)
    plsc.subcore_barrier()

    def issue_edges(k, b):
        pltpu.async_copy(edges_hbm.at[s, k], ebufs[b], se[b])
        pltpu.async_copy(w_hbm.at[s, k], wbufs[b], se[b])

    def wait_edges(k, b):
        pltpu.make_async_copy(edges_hbm.at[s, k], ebufs[b], se[b]).wait()
        pltpu.make_async_copy(w_hbm.at[s, k], wbufs[b], se[b]).wait()

    def run_layer(table, accbuf, alpha):
        def start_gather(k, b):
            pltpu.async_copy(table.at[ebufs[b].at[0]], rbufs[b], sg[b])

        def wait_gather(k, b):
            pltpu.make_async_copy(table.at[ebufs[b].at[0]],
                                  rbufs[b], sg[b]).wait()

        def mul_rows(b):
            rowsb = rbufs[b]
            eb = ebufs[b]

            def group(g, carry):
                wv = wbufs[b][pl.ds(g * 16, 16)] * alpha
                for e in range(16):
                    wgt = wv[e]
                    row = g * 16 + e
                    for j in range(DH // 16):
                        sl = pl.ds(j * 16, 16)
                        rowsb[row, sl] = rowsb[row, sl] * wgt
                return carry

            lax.fori_loop(0, CHUNK // 16, group, 0)

        def start_scatter(k, b):
            pltpu.async_copy(rbufs[b], accbuf.at[ebufs[b].at[1]], ss[b],
                             add=True)

        def wait_scatter(k, b):
            pltpu.make_async_copy(rbufs[b], accbuf.at[ebufs[b].at[1]],
                                  ss[b]).wait()

        issue_edges(0, 0)
        issue_edges(1, 1)
        wait_edges(0, 0)
        start_gather(0, 0)

        def block_body(t, carry):
            for u in range(4):
                k = 4 * t + u
                b1 = (u + 1) % 4
                b2 = (u + 2) % 4
                if u < 2:
                    @pl.when(t > 0)
                    def _s0():
                        wait_scatter(k - 2, b2)
                else:
                    wait_scatter(k - 2, b2)
                if u < 2:
                    @pl.when(k + 2 < NCH)
                    def _s1():
                        issue_edges(k + 2, b2)
                else:
                    @pl.when(t < BLOCKS - 1)
                    def _s1b():
                        issue_edges(k + 2, b2)
                if u < 3:
                    @pl.when(k + 1 < NCH)
                    def _s2():
                        wait_edges(k + 1, b1)
                        start_gather(k + 1, b1)
                else:
                    @pl.when(t < BLOCKS - 1)
                    def _s2b():
                        wait_edges(k + 1, b1)
                        start_gather(k + 1, b1)
                wait_gather(k, u)
                mul_rows(u)
                start_scatter(k, u)
            return carry

        lax.fori_loop(0, BLOCKS, block_body, 0)

        wait_scatter(NCH - 2, (NCH - 2) % 4)
        wait_scatter(NCH - 1, (NCH - 1) % 4)
        plsc.subcore_barrier()

    run_layer(bufa, bufb, alphas_v[0])
    each_slice(lambda b, n: pltpu.sync_copy(bufb.at[pl.ds(b, n)],
                                            y_hbm.at[0, c, pl.ds(b, n)]))
    each_slice(lambda b, n: pltpu.sync_copy(zeros_hbm.at[pl.ds(b, n)],
                                            bufa.at[pl.ds(b, n)]))
    plsc.subcore_barrier()

    run_layer(bufb, bufa, alphas_v[1])
    each_slice(lambda b, n: pltpu.sync_copy(bufa.at[pl.ds(b, n)],
                                            y_hbm.at[1, c, pl.ds(b, n)]))
    each_slice(lambda b, n: pltpu.sync_copy(zeros_hbm.at[pl.ds(b, n)],
                                            bufb.at[pl.ds(b, n)]))
    plsc.subcore_barrier()

    run_layer(bufa, bufb, alphas_v[2])
    each_slice(lambda b, n: pltpu.sync_copy(bufb.at[pl.ds(b, n)],
                                            y_hbm.at[2, c, pl.ds(b, n)]))


def kernel(x, edge_index, edge_weight, alphas_raw):
    alphas = jnp.tanh(alphas_raw.astype(jnp.float32))
    src = edge_index[0].astype(jnp.int32)
    dst = edge_index[1].astype(jnp.int32)
    w = edge_weight.astype(jnp.float32)

    pad = E_PAD - N_EDGES
    packed = jnp.stack([
        jnp.pad(src, (0, pad)).reshape(NS, NCH, CHUNK),
        jnp.pad(dst, (0, pad)).reshape(NS, NCH, CHUNK),
    ], axis=2)
    w_p = jnp.pad(w, (0, pad)).reshape(NS, NCH, CHUNK)
    zeros = jnp.zeros((N_NODES, DH), jnp.float32)
    alphas3 = jnp.broadcast_to(alphas[1:DEPTH + 1, None], (DEPTH, 16))
    xp = x.reshape(N_NODES, NC, DH).transpose(1, 0, 2)

    ys = _poly_conv(xp, packed, w_p, alphas3, zeros)
    ys_n = ys.transpose(0, 2, 1, 3).reshape(DEPTH, N_NODES, D_FEAT)
    return jnp.concatenate([x[:, None, :], ys_n.transpose(1, 0, 2)], axis=1)
